# Initial kernel scaffold; baseline (speedup 1.0000x reference)
import jax
import jax.numpy as jnp
from jax.experimental import pallas as pl


def kernel(x, edge_index, emb_W, emb_b, bn_gamma0, bn_beta0, gat_W0, att_src0, att_dst0, gat_b0, bn_gamma1, bn_beta1, gat_W1, att_src1, att_dst1, gat_b1):
    raise NotImplementedError("write your pallas kernel here")


# speedup vs baseline: 23.6369x; 23.6369x over previous
import functools

import jax
import jax.numpy as jnp
from jax import lax
from jax.experimental import pallas as pl
from jax.experimental.pallas import tpu as pltpu
from jax.experimental.pallas import tpu_sc as plsc

_N = 10000
_E = 320000
_D = 128

_NC = 2
_NS = 16
_NW = _NC * _NS
_EPW = _E // _NW
_CK = 80
_NCHUNK = _EPW // _CK
_NPT = 632
_NPT_PIECES = tuple((o, min(_CK, _NPT - o)) for o in range(0, _NPT, _CK))


def _bn_lrelu(hidden, gamma, beta):
    mean = jnp.mean(hidden, axis=0)
    var = jnp.mean((hidden - mean) ** 2, axis=0)
    h = (hidden - mean) / jnp.sqrt(var + 1e-5) * gamma + beta
    return jnp.where(h > 0, h, 0.01 * h)


def _pre_tail(h, W, a_s, a_d, h_o, hW_o, asrc_o, adst_o):
    hW = jnp.dot(h, W, preferred_element_type=jnp.float32)
    h_o[...] = h
    hW_o[...] = hW
    asrc_o[...] = jnp.sum(hW * a_s[None, :], axis=1)
    adst_o[...] = jnp.sum(hW * a_d[None, :], axis=1)


def _emb_pre_body(x_r, eW_r, eb_r, g_r, b_r, W_r, as_r, ad_r,
                  h_o, hW_o, asrc_o, adst_o):
    hidden = jnp.dot(x_r[...], eW_r[...], preferred_element_type=jnp.float32)
    hidden = hidden + eb_r[...][None, :]
    h = _bn_lrelu(hidden, g_r[...], b_r[...])
    _pre_tail(h, W_r[...], as_r[...], ad_r[...], h_o, hW_o, asrc_o, adst_o)


def _combine(nump, denp, asrc, adst, hW, gb, h):
    num = nump[:_N] + nump[_N:]
    den = denp[:_N] + denp[_N:]
    es = asrc + adst
    es = jnp.where(es > 0, es, 0.2 * es)
    exs = jnp.exp(es)
    num = num + exs[:, None] * hW
    den = den + exs
    return num / (den + 1e-16)[:, None] + gb[None, :] + h


def _mid_body(nump_r, denp_r, asrc_r, adst_r, hW_r, gb_r, h_r,
              g_r, b_r, W_r, as_r, ad_r,
              h_o, hW_o, asrc_o, adst_o):
    hidden = _combine(nump_r[...], denp_r[...], asrc_r[...], adst_r[...],
                      hW_r[...], gb_r[...], h_r[...])
    h = _bn_lrelu(hidden, g_r[...], b_r[...])
    _pre_tail(h, W_r[...], as_r[...], ad_r[...], h_o, hW_o, asrc_o, adst_o)


def _final_body(nump_r, denp_r, asrc_r, adst_r, hW_r, gb_r, h_r, out_o):
    out_o[...] = _combine(nump_r[...], denp_r[...], asrc_r[...], adst_r[...],
                          hW_r[...], gb_r[...], h_r[...])


_f32 = jnp.float32
_nd = jax.ShapeDtypeStruct((_N, _D), _f32)
_nv = jax.ShapeDtypeStruct((_N,), _f32)

_emb_pre = pl.pallas_call(
    _emb_pre_body, out_shape=(_nd, _nd, _nv, _nv))

_mid = pl.pallas_call(
    _mid_body, out_shape=(_nd, _nd, _nv, _nv))

_final = pl.pallas_call(
    _final_body, out_shape=_nd)


_sc_mesh = plsc.VectorSubcoreMesh(
    core_axis_name="c", subcore_axis_name="s", num_cores=_NC,
    num_subcores=_NS)


@functools.partial(
    pl.kernel,
    out_type=(jax.ShapeDtypeStruct((_NC * _N, _D), _f32),
              jax.ShapeDtypeStruct((_NC * _N,), _f32)),
    mesh=_sc_mesh,
    scratch_types=dict(
        asrc_v=pltpu.VMEM((_N,), _f32),
        adst_v=pltpu.VMEM((_N,), _f32),
        sidx=pltpu.VMEM((_CK,), jnp.int32),
        didx=pltpu.VMEM((_CK,), jnp.int32),
        rows=pltpu.VMEM((_CK, _D), _f32),
        exv=pltpu.VMEM((_CK + 16,), _f32),
        num_sh=pltpu.VMEM_SHARED((_N, _D), _f32),
        den_sh=pltpu.VMEM_SHARED((_N,), _f32),
        sem=pltpu.SemaphoreType.DMA,
    ),
    compiler_params=pltpu.CompilerParams(needs_layout_passes=False),
)
def _sc_edge(hW_hbm, asrc_hbm, adst_hbm, src_hbm, dst_hbm,
             num_out, den_out,
             asrc_v, adst_v, sidx, didx, rows, exv, num_sh, den_sh, sem):
    cid = lax.axis_index("c")
    sid = lax.axis_index("s")
    wid = sid * _NC + cid

    zero16 = jnp.zeros((16,), _f32)
    for j in range(_CK):
        for cc in range(_D // 16):
            rows[j, pl.ds(cc * 16, 16)] = zero16
    for j5 in range(_CK // 16 + 1):
        exv[pl.ds(j5 * 16, 16)] = zero16
    rbase = pl.multiple_of(jnp.minimum(sid * _NPT, _N - _NPT), 8)
    for off, sz in _NPT_PIECES:
        pltpu.sync_copy(rows.at[pl.ds(0, sz)],
                        num_sh.at[pl.ds(rbase + off, sz)])
        pltpu.sync_copy(exv.at[pl.ds(0, sz)],
                        den_sh.at[pl.ds(rbase + off, sz)])
    pltpu.sync_copy(asrc_hbm, asrc_v)
    pltpu.sync_copy(adst_hbm, adst_v)
    plsc.subcore_barrier()

    ebase = pl.multiple_of(wid * _EPW, 8)

    def chunk_body(c, carry):
        base = pl.multiple_of(ebase + c * _CK, 8)
        pltpu.sync_copy(src_hbm.at[pl.ds(base, _CK)], sidx)
        pltpu.sync_copy(dst_hbm.at[pl.ds(base, _CK)], didx)
        cp = pltpu.async_copy(hW_hbm.at[sidx], rows, sem)
        for j5 in range(_CK // 16):
            s16 = sidx[pl.ds(j5 * 16, 16)]
            d16 = didx[pl.ds(j5 * 16, 16)]
            e = (plsc.load_gather(asrc_v, [s16])
                 + plsc.load_gather(adst_v, [d16]))
            e = jnp.where(e > 0, e, 0.2 * e)
            exv[pl.ds(16 + j5 * 16, 16)] = jnp.exp(e)
        cp.wait()
        for j in range(_CK):
            w = plsc.load_gather(exv, [jnp.full((16,), 16 + j, jnp.int32)])
            for cc in range(8):
                rows[j, pl.ds(cc * 16, 16)] = rows[j, pl.ds(cc * 16, 16)] * w
        pltpu.sync_copy(rows, num_sh.at[didx], add=True)
        pltpu.sync_copy(exv.at[pl.ds(16, _CK)], den_sh.at[didx], add=True)
        return carry

    lax.fori_loop(0, _NCHUNK, chunk_body, 0)
    plsc.subcore_barrier()

    obase = pl.multiple_of(cid * _N + rbase, 8)
    for off, sz in _NPT_PIECES:
        pltpu.sync_copy(num_sh.at[pl.ds(rbase + off, sz)],
                        rows.at[pl.ds(0, sz)])
        pltpu.sync_copy(rows.at[pl.ds(0, sz)],
                        num_out.at[pl.ds(obase + off, sz)])
        pltpu.sync_copy(den_sh.at[pl.ds(rbase + off, sz)],
                        exv.at[pl.ds(0, sz)])
        pltpu.sync_copy(exv.at[pl.ds(0, sz)],
                        den_out.at[pl.ds(obase + off, sz)])


def kernel(x, edge_index, emb_W, emb_b,
           bn_gamma0, bn_beta0, gat_W0, att_src0, att_dst0, gat_b0,
           bn_gamma1, bn_beta1, gat_W1, att_src1, att_dst1, gat_b1):
    src = edge_index[0]
    dst = edge_index[1]

    h0, hW0, asrc0, adst0 = _emb_pre(
        x, emb_W, emb_b, bn_gamma0, bn_beta0, gat_W0, att_src0, att_dst0)
    num0, den0 = _sc_edge(hW0, asrc0, adst0, src, dst)
    h1, hW1, asrc1, adst1 = _mid(
        num0, den0, asrc0, adst0, hW0, gat_b0, h0,
        bn_gamma1, bn_beta1, gat_W1, att_src1, att_dst1)
    num1, den1 = _sc_edge(hW1, asrc1, adst1, src, dst)
    return _final(num1, den1, asrc1, adst1, hW1, gat_b1, h1)


# speedup vs baseline: 24.8085x; 1.0496x over previous
import functools

import jax
import jax.numpy as jnp
from jax import lax
from jax.experimental import pallas as pl
from jax.experimental.pallas import tpu as pltpu
from jax.experimental.pallas import tpu_sc as plsc

_N = 10000
_E = 320000
_D = 128

_NC = 2
_NS = 16
_NW = _NC * _NS
_EPW = _E // _NW
_CK = 80
_NCHUNK = _EPW // _CK
_NPT = 632
_NPT_PIECES = tuple((o, min(_CK, _NPT - o)) for o in range(0, _NPT, _CK))


def _bn_lrelu(hidden, gamma, beta):
    mean = jnp.mean(hidden, axis=0)
    var = jnp.mean((hidden - mean) ** 2, axis=0)
    h = (hidden - mean) / jnp.sqrt(var + 1e-5) * gamma + beta
    return jnp.where(h > 0, h, 0.01 * h)


def _pre_tail(h, W, a_s, a_d, h_o, hW_o, asrc_o, adst_o):
    hW = jnp.dot(h, W, preferred_element_type=jnp.float32)
    h_o[...] = h
    hW_o[...] = hW
    asrc_o[...] = jnp.sum(hW * a_s[None, :], axis=1)
    adst_o[...] = jnp.sum(hW * a_d[None, :], axis=1)


def _emb_pre_body(x_r, eW_r, eb_r, g_r, b_r, W_r, as_r, ad_r,
                  h_o, hW_o, asrc_o, adst_o):
    hidden = jnp.dot(x_r[...], eW_r[...], preferred_element_type=jnp.float32)
    hidden = hidden + eb_r[...][None, :]
    h = _bn_lrelu(hidden, g_r[...], b_r[...])
    _pre_tail(h, W_r[...], as_r[...], ad_r[...], h_o, hW_o, asrc_o, adst_o)


def _combine(nump, denp, asrc, adst, hW, gb, h):
    num = nump[:_N] + nump[_N:]
    den = denp[:_N] + denp[_N:]
    es = asrc + adst
    es = jnp.where(es > 0, es, 0.2 * es)
    exs = jnp.exp(es)
    num = num + exs[:, None] * hW
    den = den + exs
    return num / (den + 1e-16)[:, None] + gb[None, :] + h


def _mid_body(nump_r, denp_r, asrc_r, adst_r, hW_r, gb_r, h_r,
              g_r, b_r, W_r, as_r, ad_r,
              h_o, hW_o, asrc_o, adst_o):
    hidden = _combine(nump_r[...], denp_r[...], asrc_r[...], adst_r[...],
                      hW_r[...], gb_r[...], h_r[...])
    h = _bn_lrelu(hidden, g_r[...], b_r[...])
    _pre_tail(h, W_r[...], as_r[...], ad_r[...], h_o, hW_o, asrc_o, adst_o)


def _final_body(nump_r, denp_r, asrc_r, adst_r, hW_r, gb_r, h_r, out_o):
    out_o[...] = _combine(nump_r[...], denp_r[...], asrc_r[...], adst_r[...],
                          hW_r[...], gb_r[...], h_r[...])


_f32 = jnp.float32
_nd = jax.ShapeDtypeStruct((_N, _D), _f32)
_nv = jax.ShapeDtypeStruct((_N,), _f32)

_emb_pre = pl.pallas_call(
    _emb_pre_body, out_shape=(_nd, _nd, _nv, _nv))

_mid = pl.pallas_call(
    _mid_body, out_shape=(_nd, _nd, _nv, _nv))

_final = pl.pallas_call(
    _final_body, out_shape=_nd)


_sc_mesh = plsc.VectorSubcoreMesh(
    core_axis_name="c", subcore_axis_name="s", num_cores=_NC,
    num_subcores=_NS)


@functools.partial(
    pl.kernel,
    out_type=(jax.ShapeDtypeStruct((_NC * _N, _D), _f32),
              jax.ShapeDtypeStruct((_NC * _N,), _f32)),
    mesh=_sc_mesh,
    scratch_types=dict(
        asrc_v=pltpu.VMEM((_N,), _f32),
        adst_v=pltpu.VMEM((_N,), _f32),
        sidx_a=pltpu.VMEM((_CK,), jnp.int32),
        sidx_b=pltpu.VMEM((_CK,), jnp.int32),
        didx_a=pltpu.VMEM((_CK,), jnp.int32),
        didx_b=pltpu.VMEM((_CK,), jnp.int32),
        rows_a=pltpu.VMEM((_CK, _D), _f32),
        rows_b=pltpu.VMEM((_CK, _D), _f32),
        exv_a=pltpu.VMEM((_CK + 16,), _f32),
        exv_b=pltpu.VMEM((_CK + 16,), _f32),
        num_sh=pltpu.VMEM_SHARED((_N, _D), _f32),
        den_sh=pltpu.VMEM_SHARED((_N,), _f32),
        sem_a=pltpu.SemaphoreType.DMA,
        sem_b=pltpu.SemaphoreType.DMA,
        sem_ia=pltpu.SemaphoreType.DMA,
        sem_ib=pltpu.SemaphoreType.DMA,
    ),
    compiler_params=pltpu.CompilerParams(needs_layout_passes=False),
)
def _sc_edge(hW_hbm, asrc_hbm, adst_hbm, src_hbm, dst_hbm,
             num_out, den_out,
             asrc_v, adst_v, sidx_a, sidx_b, didx_a, didx_b,
             rows_a, rows_b, exv_a, exv_b,
             num_sh, den_sh, sem_a, sem_b, sem_ia, sem_ib):
    cid = lax.axis_index("c")
    sid = lax.axis_index("s")
    wid = sid * _NC + cid

    zero16 = jnp.zeros((16,), _f32)
    for j in range(_CK):
        for cc in range(_D // 16):
            rows_a[j, pl.ds(cc * 16, 16)] = zero16
    for j5 in range(_CK // 16 + 1):
        exv_a[pl.ds(j5 * 16, 16)] = zero16
    rbase = pl.multiple_of(jnp.minimum(sid * _NPT, _N - _NPT), 8)
    for off, sz in _NPT_PIECES:
        pltpu.sync_copy(rows_a.at[pl.ds(0, sz)],
                        num_sh.at[pl.ds(rbase + off, sz)])
        pltpu.sync_copy(exv_a.at[pl.ds(0, sz)],
                        den_sh.at[pl.ds(rbase + off, sz)])
    pltpu.sync_copy(asrc_hbm, asrc_v)
    pltpu.sync_copy(adst_hbm, adst_v)
    plsc.subcore_barrier()

    ebase = pl.multiple_of(wid * _EPW, 8)

    def pf_idx(c, sidx, didx, sem):
        base = pl.multiple_of(ebase + c * _CK, 8)
        pltpu.async_copy(src_hbm.at[pl.ds(base, _CK)], sidx, sem)
        pltpu.async_copy(dst_hbm.at[pl.ds(base, _CK)], didx, sem)

    def wait_idx(sidx, didx, sem):
        pltpu.make_async_copy(src_hbm.at[pl.ds(0, _CK)], sidx, sem).wait()
        pltpu.make_async_copy(dst_hbm.at[pl.ds(0, _CK)], didx, sem).wait()

    def compute_ex(sidx, didx, exv):
        for j5 in range(_CK // 16):
            s16 = sidx[pl.ds(j5 * 16, 16)]
            d16 = didx[pl.ds(j5 * 16, 16)]
            e = (plsc.load_gather(asrc_v, [s16])
                 + plsc.load_gather(adst_v, [d16]))
            e = jnp.where(e > 0, e, 0.2 * e)
            exv[pl.ds(16 + j5 * 16, 16)] = jnp.exp(e)

    def finish(rows, exv, didx, sidx, sem):
        pltpu.make_async_copy(hW_hbm.at[sidx], rows, sem).wait()
        for j in range(_CK):
            w = plsc.load_gather(exv, [jnp.full((16,), 16 + j, jnp.int32)])
            for cc in range(8):
                rows[j, pl.ds(cc * 16, 16)] = rows[j, pl.ds(cc * 16, 16)] * w
        pltpu.sync_copy(rows, num_sh.at[didx], add=True)
        pltpu.sync_copy(exv.at[pl.ds(16, _CK)], den_sh.at[didx], add=True)

    def start_rows(sidx, rows, sem):
        pltpu.async_copy(hW_hbm.at[sidx], rows, sem)

    pf_idx(0, sidx_a, didx_a, sem_ia)
    wait_idx(sidx_a, didx_a, sem_ia)
    start_rows(sidx_a, rows_a, sem_a)
    pf_idx(1, sidx_b, didx_b, sem_ib)

    def pair_body(c2, carry):
        ca = c2 * 2
        compute_ex(sidx_a, didx_a, exv_a)
        wait_idx(sidx_b, didx_b, sem_ib)
        start_rows(sidx_b, rows_b, sem_b)
        finish(rows_a, exv_a, didx_a, sidx_a, sem_a)
        pf_idx(ca + 2, sidx_a, didx_a, sem_ia)
        compute_ex(sidx_b, didx_b, exv_b)
        wait_idx(sidx_a, didx_a, sem_ia)
        start_rows(sidx_a, rows_a, sem_a)
        finish(rows_b, exv_b, didx_b, sidx_b, sem_b)
        pf_idx(ca + 3, sidx_b, didx_b, sem_ib)
        return carry

    lax.fori_loop(0, (_NCHUNK - 1) // 2, pair_body, 0)
    compute_ex(sidx_a, didx_a, exv_a)
    finish(rows_a, exv_a, didx_a, sidx_a, sem_a)
    wait_idx(sidx_b, didx_b, sem_ib)
    plsc.subcore_barrier()

    obase = pl.multiple_of(cid * _N + rbase, 8)
    for off, sz in _NPT_PIECES:
        pltpu.sync_copy(num_sh.at[pl.ds(rbase + off, sz)],
                        rows_a.at[pl.ds(0, sz)])
        pltpu.sync_copy(rows_a.at[pl.ds(0, sz)],
                        num_out.at[pl.ds(obase + off, sz)])
        pltpu.sync_copy(den_sh.at[pl.ds(rbase + off, sz)],
                        exv_a.at[pl.ds(0, sz)])
        pltpu.sync_copy(exv_a.at[pl.ds(0, sz)],
                        den_out.at[pl.ds(obase + off, sz)])


def kernel(x, edge_index, emb_W, emb_b,
           bn_gamma0, bn_beta0, gat_W0, att_src0, att_dst0, gat_b0,
           bn_gamma1, bn_beta1, gat_W1, att_src1, att_dst1, gat_b1):
    pad = jnp.zeros((_CK,), jnp.int32)
    src = jnp.concatenate([edge_index[0], pad])
    dst = jnp.concatenate([edge_index[1], pad])

    h0, hW0, asrc0, adst0 = _emb_pre(
        x, emb_W, emb_b, bn_gamma0, bn_beta0, gat_W0, att_src0, att_dst0)
    num0, den0 = _sc_edge(hW0, asrc0, adst0, src, dst)
    h1, hW1, asrc1, adst1 = _mid(
        num0, den0, asrc0, adst0, hW0, gat_b0, h0,
        bn_gamma1, bn_beta1, gat_W1, att_src1, att_dst1)
    num1, den1 = _sc_edge(hW1, asrc1, adst1, src, dst)
    return _final(num1, den1, asrc1, adst1, hW1, gat_b1, h1)


# speedup vs baseline: 26.8695x; 1.0831x over previous
import functools

import jax
import jax.numpy as jnp
from jax import lax
from jax.experimental import pallas as pl
from jax.experimental.pallas import tpu as pltpu
from jax.experimental.pallas import tpu_sc as plsc

_N = 10000
_E = 320000
_D = 128

_NC = 2
_NS = 16
_NW = _NC * _NS
_EPW = _E // _NW
_CK = 64
_NCHUNK = _EPW // _CK
_TAIL = _EPW - _NCHUNK * _CK
_NPT = 632
_NPT_PIECES = tuple((o, min(_CK, _NPT - o)) for o in range(0, _NPT, _CK))


def _bn_lrelu(hidden, gamma, beta):
    mean = jnp.mean(hidden, axis=0)
    var = jnp.mean((hidden - mean) ** 2, axis=0)
    h = (hidden - mean) / jnp.sqrt(var + 1e-5) * gamma + beta
    return jnp.where(h > 0, h, 0.01 * h)


def _pre_tail(h, W, a_s, a_d, h_o, hW_o, asrc_o, adst_o):
    hW = jnp.dot(h, W, preferred_element_type=jnp.float32)
    h_o[...] = h
    hW_o[...] = hW
    asrc_o[...] = jnp.sum(hW * a_s[None, :], axis=1)
    adst_o[...] = jnp.sum(hW * a_d[None, :], axis=1)


def _emb_pre_body(x_r, eW_r, eb_r, g_r, b_r, W_r, as_r, ad_r,
                  h_o, hW_o, asrc_o, adst_o):
    hidden = jnp.dot(x_r[...], eW_r[...], preferred_element_type=jnp.float32)
    hidden = hidden + eb_r[...][None, :]
    h = _bn_lrelu(hidden, g_r[...], b_r[...])
    _pre_tail(h, W_r[...], as_r[...], ad_r[...], h_o, hW_o, asrc_o, adst_o)


def _combine(nump, denp, asrc, adst, hW, gb, h):
    num = nump[:_N] + nump[_N:]
    den = denp[:_N] + denp[_N:]
    es = asrc + adst
    es = jnp.where(es > 0, es, 0.2 * es)
    exs = jnp.exp(es)
    num = num + exs[:, None] * hW
    den = den + exs
    return num / (den + 1e-16)[:, None] + gb[None, :] + h


def _mid_body(nump_r, denp_r, asrc_r, adst_r, hW_r, gb_r, h_r,
              g_r, b_r, W_r, as_r, ad_r,
              h_o, hW_o, asrc_o, adst_o):
    hidden = _combine(nump_r[...], denp_r[...], asrc_r[...], adst_r[...],
                      hW_r[...], gb_r[...], h_r[...])
    h = _bn_lrelu(hidden, g_r[...], b_r[...])
    _pre_tail(h, W_r[...], as_r[...], ad_r[...], h_o, hW_o, asrc_o, adst_o)


def _final_body(nump_r, denp_r, asrc_r, adst_r, hW_r, gb_r, h_r, out_o):
    out_o[...] = _combine(nump_r[...], denp_r[...], asrc_r[...], adst_r[...],
                          hW_r[...], gb_r[...], h_r[...])


_f32 = jnp.float32
_nd = jax.ShapeDtypeStruct((_N, _D), _f32)
_nv = jax.ShapeDtypeStruct((_N,), _f32)

_emb_pre = pl.pallas_call(
    _emb_pre_body, out_shape=(_nd, _nd, _nv, _nv))

_mid = pl.pallas_call(
    _mid_body, out_shape=(_nd, _nd, _nv, _nv))

_final = pl.pallas_call(
    _final_body, out_shape=_nd)


_sc_mesh = plsc.VectorSubcoreMesh(
    core_axis_name="c", subcore_axis_name="s", num_cores=_NC,
    num_subcores=_NS)


@functools.partial(
    pl.kernel,
    out_type=(jax.ShapeDtypeStruct((_NC * _N, _D), _f32),
              jax.ShapeDtypeStruct((_NC * _N,), _f32)),
    mesh=_sc_mesh,
    scratch_types=dict(
        asrc_v=pltpu.VMEM((_N,), _f32),
        adst_v=pltpu.VMEM((_N,), _f32),
        sidx=[pltpu.VMEM((_CK,), jnp.int32) for _ in range(3)],
        didx=[pltpu.VMEM((_CK,), jnp.int32) for _ in range(3)],
        rows=[pltpu.VMEM((_CK, _D), _f32) for _ in range(3)],
        exv=[pltpu.VMEM((_CK + 16,), _f32) for _ in range(3)],
        sidx_t=pltpu.VMEM((_TAIL,), jnp.int32),
        didx_t=pltpu.VMEM((_TAIL,), jnp.int32),
        rows_t=pltpu.VMEM((_TAIL, _D), _f32),
        exv_t=pltpu.VMEM((_TAIL + 16,), _f32),
        sem_i=[pltpu.SemaphoreType.DMA for _ in range(3)],
        sem_g=[pltpu.SemaphoreType.DMA for _ in range(3)],
        sem_s=[pltpu.SemaphoreType.DMA for _ in range(3)],
        num_sh=pltpu.VMEM_SHARED((_N, _D), _f32),
        den_sh=pltpu.VMEM_SHARED((_N,), _f32),
    ),
    compiler_params=pltpu.CompilerParams(needs_layout_passes=False),
)
def _sc_edge(hW_hbm, asrc_hbm, adst_hbm, src_hbm, dst_hbm,
             num_out, den_out,
             asrc_v, adst_v, sidx, didx, rows, exv,
             sidx_t, didx_t, rows_t, exv_t, sem_i, sem_g, sem_s,
             num_sh, den_sh):
    cid = lax.axis_index("c")
    sid = lax.axis_index("s")
    wid = sid * _NC + cid

    zero16 = jnp.zeros((16,), _f32)
    for j in range(_CK):
        for cc in range(_D // 16):
            rows[0][j, pl.ds(cc * 16, 16)] = zero16
    for j5 in range(_CK // 16 + 1):
        exv[0][pl.ds(j5 * 16, 16)] = zero16
    rbase = pl.multiple_of(jnp.minimum(sid * _NPT, _N - _NPT), 8)
    for off, sz in _NPT_PIECES:
        pltpu.sync_copy(rows[0].at[pl.ds(0, sz)],
                        num_sh.at[pl.ds(rbase + off, sz)])
        pltpu.sync_copy(exv[0].at[pl.ds(0, sz)],
                        den_sh.at[pl.ds(rbase + off, sz)])
    pltpu.sync_copy(asrc_hbm, asrc_v)
    pltpu.sync_copy(adst_hbm, adst_v)
    plsc.subcore_barrier()

    ebase = pl.multiple_of(wid * _EPW, 8)

    def pf_idx(c, k):
        base = pl.multiple_of(ebase + c * _CK, 8)
        pltpu.async_copy(src_hbm.at[pl.ds(base, _CK)], sidx[k], sem_i[k])
        pltpu.async_copy(dst_hbm.at[pl.ds(base, _CK)], didx[k], sem_i[k])

    def wait_idx(k):
        pltpu.make_async_copy(src_hbm.at[pl.ds(0, _CK)], sidx[k],
                              sem_i[k]).wait()
        pltpu.make_async_copy(dst_hbm.at[pl.ds(0, _CK)], didx[k],
                              sem_i[k]).wait()

    def start_rows(k):
        pltpu.async_copy(hW_hbm.at[sidx[k]], rows[k], sem_g[k])

    def compute_ex(k):
        for j5 in range(_CK // 16):
            s16 = sidx[k][pl.ds(j5 * 16, 16)]
            d16 = didx[k][pl.ds(j5 * 16, 16)]
            e = (plsc.load_gather(asrc_v, [s16])
                 + plsc.load_gather(adst_v, [d16]))
            e = jnp.where(e > 0, e, 0.2 * e)
            exv[k][pl.ds(16 + j5 * 16, 16)] = jnp.exp(e)

    def scale_scat(k):
        pltpu.make_async_copy(hW_hbm.at[sidx[k]], rows[k], sem_g[k]).wait()
        for j in range(_CK):
            w = plsc.load_gather(exv[k],
                                 [jnp.full((16,), 16 + j, jnp.int32)])
            for cc in range(8):
                rows[k][j, pl.ds(cc * 16, 16)] = (
                    rows[k][j, pl.ds(cc * 16, 16)] * w)
        pltpu.async_copy(rows[k], num_sh.at[didx[k]], sem_s[k], add=True)
        pltpu.async_copy(exv[k].at[pl.ds(16, _CK)], den_sh.at[didx[k]],
                         sem_s[k], add=True)

    def wait_scat(k):
        pltpu.make_async_copy(rows[k], num_sh.at[didx[k]],
                              sem_s[k]).wait()
        pltpu.make_async_copy(exv[k].at[pl.ds(16, _CK)],
                              den_sh.at[didx[k]], sem_s[k]).wait()

    def slot(c, k, first):
        k1, k2 = (k + 1) % 3, (k + 2) % 3
        wait_idx(k1)
        start_rows(k1)
        compute_ex(k)
        scale_scat(k)
        if not first:
            wait_scat(k2)
        pf_idx(c + 2, k2)

    pf_idx(0, 0)
    pf_idx(1, 1)
    wait_idx(0)
    start_rows(0)
    slot(0, 0, True)
    def tri_body(t, carry):
        c = 1 + 3 * t
        slot(c, 1, False)
        slot(c + 1, 2, False)
        slot(c + 2, 0, False)
        return carry
    lax.fori_loop(0, (_NCHUNK - 3) // 3, tri_body, 0)
    slot(_NCHUNK - 2, 1, False)
    wait_idx(0)
    compute_ex(2)
    scale_scat(2)
    wait_scat(1)
    wait_scat(2)

    tbase = pl.multiple_of(ebase + _NCHUNK * _CK, 8)
    pltpu.sync_copy(src_hbm.at[pl.ds(tbase, _TAIL)], sidx_t)
    pltpu.sync_copy(dst_hbm.at[pl.ds(tbase, _TAIL)], didx_t)
    cpt = pltpu.async_copy(hW_hbm.at[sidx_t], rows_t, sem_g[0])
    s16 = sidx_t[pl.ds(0, 16)]
    d16 = didx_t[pl.ds(0, 16)]
    e = plsc.load_gather(asrc_v, [s16]) + plsc.load_gather(adst_v, [d16])
    e = jnp.where(e > 0, e, 0.2 * e)
    exv_t[pl.ds(16, 16)] = jnp.exp(e)
    cpt.wait()
    for j in range(_TAIL):
        w = plsc.load_gather(exv_t, [jnp.full((16,), 16 + j, jnp.int32)])
        for cc in range(8):
            rows_t[j, pl.ds(cc * 16, 16)] = rows_t[j, pl.ds(cc * 16, 16)] * w
    pltpu.sync_copy(rows_t, num_sh.at[didx_t], add=True)
    pltpu.sync_copy(exv_t.at[pl.ds(16, _TAIL)], den_sh.at[didx_t], add=True)
    plsc.subcore_barrier()

    obase = pl.multiple_of(cid * _N + rbase, 8)
    for off, sz in _NPT_PIECES:
        pltpu.sync_copy(num_sh.at[pl.ds(rbase + off, sz)],
                        rows[0].at[pl.ds(0, sz)])
        pltpu.sync_copy(rows[0].at[pl.ds(0, sz)],
                        num_out.at[pl.ds(obase + off, sz)])
        pltpu.sync_copy(den_sh.at[pl.ds(rbase + off, sz)],
                        exv[0].at[pl.ds(0, sz)])
        pltpu.sync_copy(exv[0].at[pl.ds(0, sz)],
                        den_out.at[pl.ds(obase + off, sz)])


def kernel(x, edge_index, emb_W, emb_b,
           bn_gamma0, bn_beta0, gat_W0, att_src0, att_dst0, gat_b0,
           bn_gamma1, bn_beta1, gat_W1, att_src1, att_dst1, gat_b1):
    pad = jnp.zeros((_CK,), jnp.int32)
    src = jnp.concatenate([edge_index[0], pad])
    dst = jnp.concatenate([edge_index[1], pad])

    h0, hW0, asrc0, adst0 = _emb_pre(
        x, emb_W, emb_b, bn_gamma0, bn_beta0, gat_W0, att_src0, att_dst0)
    num0, den0 = _sc_edge(hW0, asrc0, adst0, src, dst)
    h1, hW1, asrc1, adst1 = _mid(
        num0, den0, asrc0, adst0, hW0, gat_b0, h0,
        bn_gamma1, bn_beta1, gat_W1, att_src1, att_dst1)
    num1, den1 = _sc_edge(hW1, asrc1, adst1, src, dst)
    return _final(num1, den1, asrc1, adst1, hW1, gat_b1, h1)


# speedup vs baseline: 39.0208x; 1.4522x over previous
import functools

import jax
import jax.numpy as jnp
from jax import lax
from jax.experimental import pallas as pl
from jax.experimental.pallas import tpu as pltpu
from jax.experimental.pallas import tpu_sc as plsc

_N = 10000
_E = 320000
_D = 128

_NC = 2
_NS = 16
_NW = _NC * _NS
_EPW = _E // _NW
_CK = 64
_NCHUNK = _EPW // _CK
_TAIL = _EPW - _NCHUNK * _CK
_NPT = 632
_NPT_PIECES = tuple((o, min(_CK, _NPT - o)) for o in range(0, _NPT, _CK))


def _bn_lrelu(hidden, gamma, beta):
    mean = jnp.mean(hidden, axis=0)
    var = jnp.mean((hidden - mean) ** 2, axis=0)
    h = (hidden - mean) / jnp.sqrt(var + 1e-5) * gamma + beta
    return jnp.where(h > 0, h, 0.01 * h)


def _pre_tail(h, W, a_s, a_d, h_o, hW_o, asrc_o, adst_o):
    hW = jnp.dot(h, W, preferred_element_type=jnp.float32)
    h_o[...] = h
    hW_o[...] = hW
    asrc_o[...] = jnp.sum(hW * a_s[None, :], axis=1)
    adst_o[...] = jnp.sum(hW * a_d[None, :], axis=1)


def _emb_pre_body(x_r, eW_r, eb_r, g_r, b_r, W_r, as_r, ad_r,
                  h_o, hW_o, asrc_o, adst_o):
    hidden = jnp.dot(x_r[...], eW_r[...], preferred_element_type=jnp.float32)
    hidden = hidden + eb_r[...][None, :]
    h = _bn_lrelu(hidden, g_r[...], b_r[...])
    _pre_tail(h, W_r[...], as_r[...], ad_r[...], h_o, hW_o, asrc_o, adst_o)


def _combine(nump, denp, asrc, adst, hW, gb, h):
    num = nump[:_N] + nump[_N:]
    den = denp[:_N] + denp[_N:]
    es = asrc + adst
    es = jnp.where(es > 0, es, 0.2 * es)
    exs = jnp.exp(es)
    num = num + exs[:, None] * hW
    den = den + exs
    return num / (den + 1e-16)[:, None] + gb[None, :] + h


def _mid_body(nump_r, denp_r, asrc_r, adst_r, hW_r, gb_r, h_r,
              g_r, b_r, W_r, as_r, ad_r,
              h_o, hW_o, asrc_o, adst_o):
    hidden = _combine(nump_r[...], denp_r[...], asrc_r[...], adst_r[...],
                      hW_r[...], gb_r[...], h_r[...])
    h = _bn_lrelu(hidden, g_r[...], b_r[...])
    _pre_tail(h, W_r[...], as_r[...], ad_r[...], h_o, hW_o, asrc_o, adst_o)


def _final_body(nump_r, denp_r, asrc_r, adst_r, hW_r, gb_r, h_r, out_o):
    out_o[...] = _combine(nump_r[...], denp_r[...], asrc_r[...], adst_r[...],
                          hW_r[...], gb_r[...], h_r[...])


_f32 = jnp.float32
_nd = jax.ShapeDtypeStruct((_N, _D), _f32)
_nv = jax.ShapeDtypeStruct((_N,), _f32)

_emb_pre = pl.pallas_call(
    _emb_pre_body, out_shape=(_nd, _nd, _nv, _nv))

_mid = pl.pallas_call(
    _mid_body, out_shape=(_nd, _nd, _nv, _nv))

_final = pl.pallas_call(
    _final_body, out_shape=_nd)


_sc_mesh = plsc.VectorSubcoreMesh(
    core_axis_name="c", subcore_axis_name="s", num_cores=_NC,
    num_subcores=_NS)


@functools.partial(
    pl.kernel,
    out_type=(jax.ShapeDtypeStruct((_NC * _N, _D), _f32),
              jax.ShapeDtypeStruct((_NC * _N,), _f32)),
    mesh=_sc_mesh,
    scratch_types=dict(
        asrc_v=pltpu.VMEM((_N,), _f32),
        adst_v=pltpu.VMEM((_N,), _f32),
        sidx=[pltpu.VMEM((_CK,), jnp.int32) for _ in range(3)],
        didx=[pltpu.VMEM((_CK,), jnp.int32) for _ in range(3)],
        rows=[pltpu.VMEM((_CK, _D), _f32) for _ in range(3)],
        exv=[pltpu.VMEM((_CK + 16,), _f32) for _ in range(3)],
        sidx_t=pltpu.VMEM((_TAIL,), jnp.int32),
        didx_t=pltpu.VMEM((_TAIL,), jnp.int32),
        rows_t=pltpu.VMEM((_TAIL, _D), _f32),
        exv_t=pltpu.VMEM((_TAIL + 16,), _f32),
        sem_i=[pltpu.SemaphoreType.DMA for _ in range(3)],
        sem_g=[pltpu.SemaphoreType.DMA for _ in range(3)],
        sem_s=[pltpu.SemaphoreType.DMA for _ in range(3)],
        num_sh=pltpu.VMEM_SHARED((_N, _D), _f32),
        den_sh=pltpu.VMEM_SHARED((_N,), _f32),
    ),
    compiler_params=pltpu.CompilerParams(needs_layout_passes=False),
)
def _sc_edge(hW_hbm, asrc_hbm, adst_hbm, src_hbm, dst_hbm,
             num_out, den_out,
             asrc_v, adst_v, sidx, didx, rows, exv,
             sidx_t, didx_t, rows_t, exv_t, sem_i, sem_g, sem_s,
             num_sh, den_sh):
    cid = lax.axis_index("c")
    sid = lax.axis_index("s")
    wid = sid * _NC + cid

    zero16 = jnp.zeros((16,), _f32)
    for j in range(_CK):
        for cc in range(_D // 16):
            rows[0][j, pl.ds(cc * 16, 16)] = zero16
    for j5 in range(_CK // 16 + 1):
        exv[0][pl.ds(j5 * 16, 16)] = zero16
    rbase = pl.multiple_of(jnp.minimum(sid * _NPT, _N - _NPT), 8)
    for off, sz in _NPT_PIECES:
        pltpu.sync_copy(rows[0].at[pl.ds(0, sz)],
                        num_sh.at[pl.ds(rbase + off, sz)])
        pltpu.sync_copy(exv[0].at[pl.ds(0, sz)],
                        den_sh.at[pl.ds(rbase + off, sz)])
    pltpu.sync_copy(asrc_hbm, asrc_v)
    pltpu.sync_copy(adst_hbm, adst_v)
    plsc.subcore_barrier()

    ebase = pl.multiple_of(wid * _EPW, 8)

    def pf_idx(c, k):
        base = pl.multiple_of(ebase + c * _CK, 8)
        pltpu.async_copy(src_hbm.at[pl.ds(base, _CK)], sidx[k], sem_i[k])
        pltpu.async_copy(dst_hbm.at[pl.ds(base, _CK)], didx[k], sem_i[k])

    def wait_idx(k):
        pltpu.make_async_copy(src_hbm.at[pl.ds(0, _CK)], sidx[k],
                              sem_i[k]).wait()
        pltpu.make_async_copy(dst_hbm.at[pl.ds(0, _CK)], didx[k],
                              sem_i[k]).wait()

    def start_rows(k):
        pltpu.async_copy(hW_hbm.at[sidx[k]], rows[k], sem_g[k])

    def compute_ex(k):
        for j5 in range(_CK // 16):
            s16 = sidx[k][pl.ds(j5 * 16, 16)]
            d16 = didx[k][pl.ds(j5 * 16, 16)]
            e = (plsc.load_gather(asrc_v, [s16])
                 + plsc.load_gather(adst_v, [d16]))
            e = jnp.where(e > 0, e, 0.2 * e)
            exv[k][pl.ds(16 + j5 * 16, 16)] = jnp.exp(e)

    def scale_scat(k):
        pltpu.make_async_copy(hW_hbm.at[sidx[k]], rows[k], sem_g[k]).wait()
        for j5 in range(_CK // 16):
            w16 = exv[k][pl.ds(16 + j5 * 16, 16)]
            for l in range(16):
                j = j5 * 16 + l
                w = jnp.full((16,), w16[l], _f32)
                for cc in range(8):
                    rows[k][j, pl.ds(cc * 16, 16)] = (
                        rows[k][j, pl.ds(cc * 16, 16)] * w)
        pltpu.async_copy(rows[k], num_sh.at[didx[k]], sem_s[k], add=True)
        pltpu.async_copy(exv[k].at[pl.ds(16, _CK)], den_sh.at[didx[k]],
                         sem_s[k], add=True)

    def wait_scat(k):
        pltpu.make_async_copy(rows[k], num_sh.at[didx[k]],
                              sem_s[k]).wait()
        pltpu.make_async_copy(exv[k].at[pl.ds(16, _CK)],
                              den_sh.at[didx[k]], sem_s[k]).wait()

    def slot(c, k, first):
        k1, k2 = (k + 1) % 3, (k + 2) % 3
        wait_idx(k1)
        start_rows(k1)
        compute_ex(k)
        scale_scat(k)
        if not first:
            wait_scat(k2)
        pf_idx(c + 2, k2)

    pf_idx(0, 0)
    pf_idx(1, 1)
    wait_idx(0)
    start_rows(0)
    slot(0, 0, True)
    def tri_body(t, carry):
        c = 1 + 3 * t
        slot(c, 1, False)
        slot(c + 1, 2, False)
        slot(c + 2, 0, False)
        return carry
    lax.fori_loop(0, (_NCHUNK - 3) // 3, tri_body, 0)
    slot(_NCHUNK - 2, 1, False)
    wait_idx(0)
    compute_ex(2)
    scale_scat(2)
    wait_scat(1)
    wait_scat(2)

    tbase = pl.multiple_of(ebase + _NCHUNK * _CK, 8)
    pltpu.sync_copy(src_hbm.at[pl.ds(tbase, _TAIL)], sidx_t)
    pltpu.sync_copy(dst_hbm.at[pl.ds(tbase, _TAIL)], didx_t)
    cpt = pltpu.async_copy(hW_hbm.at[sidx_t], rows_t, sem_g[0])
    s16 = sidx_t[pl.ds(0, 16)]
    d16 = didx_t[pl.ds(0, 16)]
    e = plsc.load_gather(asrc_v, [s16]) + plsc.load_gather(adst_v, [d16])
    e = jnp.where(e > 0, e, 0.2 * e)
    exv_t[pl.ds(16, 16)] = jnp.exp(e)
    cpt.wait()
    for j in range(_TAIL):
        w = plsc.load_gather(exv_t, [jnp.full((16,), 16 + j, jnp.int32)])
        for cc in range(8):
            rows_t[j, pl.ds(cc * 16, 16)] = rows_t[j, pl.ds(cc * 16, 16)] * w
    pltpu.sync_copy(rows_t, num_sh.at[didx_t], add=True)
    pltpu.sync_copy(exv_t.at[pl.ds(16, _TAIL)], den_sh.at[didx_t], add=True)
    plsc.subcore_barrier()

    obase = pl.multiple_of(cid * _N + rbase, 8)
    for off, sz in _NPT_PIECES:
        pltpu.sync_copy(num_sh.at[pl.ds(rbase + off, sz)],
                        rows[0].at[pl.ds(0, sz)])
        pltpu.sync_copy(rows[0].at[pl.ds(0, sz)],
                        num_out.at[pl.ds(obase + off, sz)])
        pltpu.sync_copy(den_sh.at[pl.ds(rbase + off, sz)],
                        exv[0].at[pl.ds(0, sz)])
        pltpu.sync_copy(exv[0].at[pl.ds(0, sz)],
                        den_out.at[pl.ds(obase + off, sz)])


def kernel(x, edge_index, emb_W, emb_b,
           bn_gamma0, bn_beta0, gat_W0, att_src0, att_dst0, gat_b0,
           bn_gamma1, bn_beta1, gat_W1, att_src1, att_dst1, gat_b1):
    pad = jnp.zeros((_CK,), jnp.int32)
    src = jnp.concatenate([edge_index[0], pad])
    dst = jnp.concatenate([edge_index[1], pad])

    h0, hW0, asrc0, adst0 = _emb_pre(
        x, emb_W, emb_b, bn_gamma0, bn_beta0, gat_W0, att_src0, att_dst0)
    num0, den0 = _sc_edge(hW0, asrc0, adst0, src, dst)
    h1, hW1, asrc1, adst1 = _mid(
        num0, den0, asrc0, adst0, hW0, gat_b0, h0,
        bn_gamma1, bn_beta1, gat_W1, att_src1, att_dst1)
    num1, den1 = _sc_edge(hW1, asrc1, adst1, src, dst)
    return _final(num1, den1, asrc1, adst1, hW1, gat_b1, h1)
